# baseline (speedup 1.0000x reference)
import functools

import jax
import jax.numpy as jnp
from jax import lax
from jax.experimental import pallas as pl
from jax.experimental.pallas import tpu as pltpu
from jax.experimental.pallas import tpu_sc as plsc

N = 10000
D = 128
E = 320000
NG = 64

NC = 2
NS = 16
NW = NC * NS
C = 64
N_PAD = 10240
GC = 32
NBUF = 2
NG_FAST = 8
NG_SLOW = 2
CH_FAST = NG_FAST * GC
CH_SLOW = NG_SLOW * GC
E_PAD = NS * C * (CH_FAST + CH_SLOW)
RPS = N_PAD // NS


def _segsum_body(h_hbm, src_hbm, dst_hbm, out_hbm,
                 idx_s, idx_d, rows, acc,
                 gsem0, gsem1, gsem2, gsem3, ssem0, ssem1, ssem2, ssem3):
    c = lax.axis_index("c")
    s = lax.axis_index("s")
    wid = c * NS + s
    gsems = (gsem0, gsem1, gsem2, gsem3)
    ssems = (ssem0, ssem1, ssem2, ssem3)

    def _z(k, carry):
        i = k // (D // 16)
        j = k % (D // 16)
        rows[0, i, pl.ds(j * 16, 16)] = jnp.zeros((16,), jnp.float32)
        return carry
    lax.fori_loop(0, C * (D // 16), _z, 0)
    for r in range(RPS // C):
        pltpu.sync_copy(rows.at[0], acc.at[pl.ds(s * RPS + r * C, C)])
    plsc.subcore_barrier()

    def _startg(b, j):
        pltpu.async_copy(h_hbm.at[idx_s.at[j]], rows.at[b], gsems[b])

    def _waitg(b, j):
        pltpu.make_async_copy(h_hbm.at[idx_s.at[j]], rows.at[b], gsems[b]).wait()

    def _starts(b, j):
        pltpu.async_copy(rows.at[b], acc.at[idx_d.at[j]], ssems[b], add=True)

    def _waits(b, j):
        pltpu.make_async_copy(rows.at[b], acc.at[idx_d.at[j]], ssems[b]).wait()

    @pl.loop(0, jnp.where(c == 0, NG_FAST, NG_SLOW))
    def _group(g):
        pltpu.sync_copy(src_hbm.at[wid, pl.ds(g * GC, GC)], idx_s)
        pltpu.sync_copy(dst_hbm.at[wid, pl.ds(g * GC, GC)], idx_d)
        for b in range(NBUF):
            _startg(b, b)

        @pl.loop(0, GC - NBUF, step=NBUF)
        def _chunk(j):
            for b in range(NBUF):
                _waitg(b, j + b)
                _starts(b, j + b)
            for b in range(NBUF):
                _waits(b, j + b)
                _startg(b, j + b + NBUF)

        for b in range(NBUF):
            _waitg(b, GC - NBUF + b)
            _starts(b, GC - NBUF + b)
        for b in range(NBUF):
            _waits(b, GC - NBUF + b)

    plsc.subcore_barrier()

    pltpu.sync_copy(acc.at[pl.ds(s * RPS, RPS)],
                    out_hbm.at[c, pl.ds(s * RPS, RPS)])


_segsum = pl.kernel(
    _segsum_body,
    out_type=jax.ShapeDtypeStruct((NC, N_PAD, D), jnp.float32),
    mesh=plsc.VectorSubcoreMesh(core_axis_name="c", subcore_axis_name="s"),
    scratch_types=[
        pltpu.VMEM((GC, C), jnp.int32),
        pltpu.VMEM((GC, C), jnp.int32),
        pltpu.VMEM((NBUF, C, D), jnp.float32),
        pltpu.VMEM_SHARED((N_PAD, D), jnp.float32),
    ] + [pltpu.SemaphoreType.DMA] * 8,
)


def _bn(z, g, b):
    m = jnp.mean(z, axis=0, keepdims=True)
    zc = z - m
    v = jnp.mean(zc * zc, axis=0, keepdims=True)
    return g * zc * jax.lax.rsqrt(v + 1e-5) + b


def _mlp(h_ref, p_ref, W1_ref, b1_ref, g1_ref, be1_ref, W2_ref, b2_ref,
         gbn_ref, bbn_ref, final_relu):
    z = h_ref[...] + p_ref[0, :N, :] + p_ref[1, :N, :]
    z = jnp.dot(z, W1_ref[...], preferred_element_type=jnp.float32) + b1_ref[...]
    z = _bn(z, g1_ref[...], be1_ref[...])
    z = jnp.maximum(z, 0.0)
    z = jnp.dot(z, W2_ref[...], preferred_element_type=jnp.float32) + b2_ref[...]
    z = _bn(z, gbn_ref[...], bbn_ref[...])
    if final_relu:
        z = jnp.maximum(z, 0.0)
    return z


def _dense0_body(h_ref, p_ref, W1_ref, b1_ref, g1_ref, be1_ref,
                 W2_ref, b2_ref, gbn_ref, bbn_ref, out_ref):
    out_ref[...] = _mlp(h_ref, p_ref, W1_ref, b1_ref, g1_ref, be1_ref,
                        W2_ref, b2_ref, gbn_ref, bbn_ref, final_relu=True)


def _dense1_body(idx_ref, h_ref, p_ref, W1_ref, b1_ref, g1_ref, be1_ref,
                 W2_ref, b2_ref, gbn_ref, bbn_ref, out_ref, sel_ref):
    out_ref[...] = _mlp(h_ref, p_ref, W1_ref, b1_ref, g1_ref, be1_ref,
                        W2_ref, b2_ref, gbn_ref, bbn_ref, final_relu=False)

    def _sel(j, carry):
        r = idx_ref[j]
        sel_ref[pl.ds(j, 1), :] = out_ref[pl.ds(r, 1), :]
        return carry
    lax.fori_loop(0, NG, _sel, 0)


_VSPEC = pl.BlockSpec(memory_space=pltpu.MemorySpace.VMEM)

_dense0 = pl.pallas_call(
    _dense0_body,
    out_shape=jax.ShapeDtypeStruct((N, D), jnp.float32),
    in_specs=[_VSPEC] * 10,
    out_specs=_VSPEC,
)

_dense1 = pl.pallas_call(
    _dense1_body,
    out_shape=(jax.ShapeDtypeStruct((N, D), jnp.float32),
               jax.ShapeDtypeStruct((NG, D), jnp.float32)),
    in_specs=[pl.BlockSpec(memory_space=pltpu.MemorySpace.SMEM)] + [_VSPEC] * 10,
    out_specs=(_VSPEC, _VSPEC),
)


def kernel(x, edge_index, edge_attr, batch, index,
           W1_0, b1_0, g1_0, be1_0, W2_0, b2_0, gbn_0, bbn_0,
           W1_1, b1_1, g1_1, be1_1, W2_1, b2_1, gbn_1, bbn_1):
    src = edge_index[0]
    dst = edge_index[1]
    pad = E_PAD - E
    sink = N + (jnp.arange(pad, dtype=jnp.int32) % (N_PAD - N))
    src_f = jnp.concatenate([src, jnp.zeros((pad,), jnp.int32)])
    dst_f = jnp.concatenate([dst, sink])
    nfast = NS * CH_FAST * C

    def _split(e):
        heavy = e[:nfast].reshape(NS, CH_FAST, C)
        if CH_SLOW:
            light = e[nfast:].reshape(NS, CH_SLOW, C)
            light = jnp.pad(light, ((0, 0), (0, CH_FAST - CH_SLOW), (0, 0)))
        else:
            light = jnp.zeros_like(heavy)
        return jnp.concatenate([heavy, light], axis=0)

    src_p = _split(src_f)
    dst_p = _split(dst_f)

    r1 = lambda a: a.reshape(1, D)
    p0 = _segsum(x, src_p, dst_p)
    h1 = _dense0(x, p0, W1_0, r1(b1_0), r1(g1_0), r1(be1_0),
                 W2_0, r1(b2_0), r1(gbn_0), r1(bbn_0))
    p1 = _segsum(h1, src_p, dst_p)
    h2, sel = _dense1(index, h1, p1, W1_1, r1(b1_1), r1(g1_1), r1(be1_1),
                      W2_1, r1(b2_1), r1(gbn_1), r1(bbn_1))
    return (h2, sel)


# speedup vs baseline: 1.0665x; 1.0665x over previous
import functools

import jax
import jax.numpy as jnp
from jax import lax
from jax.experimental import pallas as pl
from jax.experimental.pallas import tpu as pltpu
from jax.experimental.pallas import tpu_sc as plsc

N = 10000
D = 128
E = 320000
NG = 64

NC = 2
NS = 16
NW = NC * NS
C = 64
N_PAD = 10240
GC = 32
NBUF = 4
NG_FAST = 9
NG_SLOW = 1
CH_FAST = NG_FAST * GC
CH_SLOW = NG_SLOW * GC
E_PAD = NS * C * (CH_FAST + CH_SLOW)
RPS = N_PAD // NS


def _segsum_body(h_hbm, src_hbm, dst_hbm, out_hbm,
                 idx_s, idx_d, rows, acc,
                 gsem0, gsem1, gsem2, gsem3, ssem0, ssem1, ssem2, ssem3):
    c = lax.axis_index("c")
    s = lax.axis_index("s")
    wid = c * NS + s
    gsems = (gsem0, gsem1, gsem2, gsem3)
    ssems = (ssem0, ssem1, ssem2, ssem3)

    def _z(k, carry):
        i = k // (D // 16)
        j = k % (D // 16)
        rows[0, i, pl.ds(j * 16, 16)] = jnp.zeros((16,), jnp.float32)
        return carry
    lax.fori_loop(0, C * (D // 16), _z, 0)
    for r in range(RPS // C):
        pltpu.sync_copy(rows.at[0], acc.at[pl.ds(s * RPS + r * C, C)])
    plsc.subcore_barrier()

    def _startg(b, j):
        pltpu.async_copy(h_hbm.at[idx_s.at[j]], rows.at[b], gsems[b])

    def _waitg(b, j):
        pltpu.make_async_copy(h_hbm.at[idx_s.at[j]], rows.at[b], gsems[b]).wait()

    def _starts(b, j):
        pltpu.async_copy(rows.at[b], acc.at[idx_d.at[j]], ssems[b], add=True)

    def _waits(b, j):
        pltpu.make_async_copy(rows.at[b], acc.at[idx_d.at[j]], ssems[b]).wait()

    @pl.loop(0, jnp.where(c == 0, NG_FAST, NG_SLOW))
    def _group(g):
        pltpu.sync_copy(src_hbm.at[wid, pl.ds(g * GC, GC)], idx_s)
        pltpu.sync_copy(dst_hbm.at[wid, pl.ds(g * GC, GC)], idx_d)
        for b in range(NBUF):
            _startg(b, b)

        @pl.loop(0, GC - NBUF, step=NBUF)
        def _chunk(j):
            for b in range(NBUF):
                _waitg(b, j + b)
                _starts(b, j + b)
            for b in range(NBUF):
                _waits(b, j + b)
                _startg(b, j + b + NBUF)

        for b in range(NBUF):
            _waitg(b, GC - NBUF + b)
            _starts(b, GC - NBUF + b)
        for b in range(NBUF):
            _waits(b, GC - NBUF + b)

    plsc.subcore_barrier()

    pltpu.sync_copy(acc.at[pl.ds(s * RPS, RPS)],
                    out_hbm.at[c, pl.ds(s * RPS, RPS)])


_segsum = pl.kernel(
    _segsum_body,
    out_type=jax.ShapeDtypeStruct((NC, N_PAD, D), jnp.float32),
    mesh=plsc.VectorSubcoreMesh(core_axis_name="c", subcore_axis_name="s"),
    scratch_types=[
        pltpu.VMEM((GC, C), jnp.int32),
        pltpu.VMEM((GC, C), jnp.int32),
        pltpu.VMEM((NBUF, C, D), jnp.float32),
        pltpu.VMEM_SHARED((N_PAD, D), jnp.float32),
    ] + [pltpu.SemaphoreType.DMA] * 8,
)


def _bn(z, g, b):
    m = jnp.mean(z, axis=0, keepdims=True)
    zc = z - m
    v = jnp.mean(zc * zc, axis=0, keepdims=True)
    return g * zc * jax.lax.rsqrt(v + 1e-5) + b


def _mlp(h_ref, p_ref, W1_ref, b1_ref, g1_ref, be1_ref, W2_ref, b2_ref,
         gbn_ref, bbn_ref, final_relu):
    z = h_ref[...] + p_ref[0, :N, :] + p_ref[1, :N, :]
    z = jnp.dot(z, W1_ref[...], preferred_element_type=jnp.float32) + b1_ref[...]
    z = _bn(z, g1_ref[...], be1_ref[...])
    z = jnp.maximum(z, 0.0)
    z = jnp.dot(z, W2_ref[...], preferred_element_type=jnp.float32) + b2_ref[...]
    z = _bn(z, gbn_ref[...], bbn_ref[...])
    if final_relu:
        z = jnp.maximum(z, 0.0)
    return z


def _dense0_body(h_ref, p_ref, W1_ref, b1_ref, g1_ref, be1_ref,
                 W2_ref, b2_ref, gbn_ref, bbn_ref, out_ref):
    out_ref[...] = _mlp(h_ref, p_ref, W1_ref, b1_ref, g1_ref, be1_ref,
                        W2_ref, b2_ref, gbn_ref, bbn_ref, final_relu=True)


def _dense1_body(idx_ref, h_ref, p_ref, W1_ref, b1_ref, g1_ref, be1_ref,
                 W2_ref, b2_ref, gbn_ref, bbn_ref, out_ref, sel_ref):
    out_ref[...] = _mlp(h_ref, p_ref, W1_ref, b1_ref, g1_ref, be1_ref,
                        W2_ref, b2_ref, gbn_ref, bbn_ref, final_relu=False)

    def _sel(j, carry):
        r = idx_ref[j]
        sel_ref[pl.ds(j, 1), :] = out_ref[pl.ds(r, 1), :]
        return carry
    lax.fori_loop(0, NG, _sel, 0)


_VSPEC = pl.BlockSpec(memory_space=pltpu.MemorySpace.VMEM)

_dense0 = pl.pallas_call(
    _dense0_body,
    out_shape=jax.ShapeDtypeStruct((N, D), jnp.float32),
    in_specs=[_VSPEC] * 10,
    out_specs=_VSPEC,
)

_dense1 = pl.pallas_call(
    _dense1_body,
    out_shape=(jax.ShapeDtypeStruct((N, D), jnp.float32),
               jax.ShapeDtypeStruct((NG, D), jnp.float32)),
    in_specs=[pl.BlockSpec(memory_space=pltpu.MemorySpace.SMEM)] + [_VSPEC] * 10,
    out_specs=(_VSPEC, _VSPEC),
)


def kernel(x, edge_index, edge_attr, batch, index,
           W1_0, b1_0, g1_0, be1_0, W2_0, b2_0, gbn_0, bbn_0,
           W1_1, b1_1, g1_1, be1_1, W2_1, b2_1, gbn_1, bbn_1):
    src = edge_index[0]
    dst = edge_index[1]
    pad = E_PAD - E
    sink = N + (jnp.arange(pad, dtype=jnp.int32) % (N_PAD - N))
    src_f = jnp.concatenate([src, jnp.zeros((pad,), jnp.int32)])
    dst_f = jnp.concatenate([dst, sink])
    nfast = NS * CH_FAST * C

    def _split(e):
        heavy = e[:nfast].reshape(NS, CH_FAST, C)
        if CH_SLOW:
            light = e[nfast:].reshape(NS, CH_SLOW, C)
            light = jnp.pad(light, ((0, 0), (0, CH_FAST - CH_SLOW), (0, 0)))
        else:
            light = jnp.zeros_like(heavy)
        return jnp.concatenate([heavy, light], axis=0)

    src_p = _split(src_f)
    dst_p = _split(dst_f)

    r1 = lambda a: a.reshape(1, D)
    p0 = _segsum(x, src_p, dst_p)
    h1 = _dense0(x, p0, W1_0, r1(b1_0), r1(g1_0), r1(be1_0),
                 W2_0, r1(b2_0), r1(gbn_0), r1(bbn_0))
    p1 = _segsum(h1, src_p, dst_p)
    h2, sel = _dense1(index, h1, p1, W1_1, r1(b1_1), r1(g1_1), r1(be1_1),
                      W2_1, r1(b2_1), r1(gbn_1), r1(bbn_1))
    return (h2, sel)


# speedup vs baseline: 1.0667x; 1.0002x over previous
import functools

import jax
import jax.numpy as jnp
from jax import lax
from jax.experimental import pallas as pl
from jax.experimental.pallas import tpu as pltpu
from jax.experimental.pallas import tpu_sc as plsc

N = 10000
D = 128
E = 320000
NG = 64

NC = 2
NS = 16
NW = NC * NS
C = 64
N_PAD = 10240
GC = 16
NBUF = 4
NG_FAST = 19
NG_SLOW = 1
CH_FAST = NG_FAST * GC
CH_SLOW = NG_SLOW * GC
E_PAD = NS * C * (CH_FAST + CH_SLOW)
RPS = N_PAD // NS


def _segsum_body(h_hbm, src_hbm, dst_hbm, out_hbm,
                 idx_s, idx_d, rows, acc,
                 gsem0, gsem1, gsem2, gsem3, ssem0, ssem1, ssem2, ssem3):
    c = lax.axis_index("c")
    s = lax.axis_index("s")
    wid = c * NS + s
    gsems = (gsem0, gsem1, gsem2, gsem3)
    ssems = (ssem0, ssem1, ssem2, ssem3)

    def _z(k, carry):
        i = k // (D // 16)
        j = k % (D // 16)
        rows[0, i, pl.ds(j * 16, 16)] = jnp.zeros((16,), jnp.float32)
        return carry
    lax.fori_loop(0, C * (D // 16), _z, 0)
    for r in range(RPS // C):
        pltpu.sync_copy(rows.at[0], acc.at[pl.ds(s * RPS + r * C, C)])
    plsc.subcore_barrier()

    def _startg(b, j):
        pltpu.async_copy(h_hbm.at[idx_s.at[j]], rows.at[b], gsems[b])

    def _waitg(b, j):
        pltpu.make_async_copy(h_hbm.at[idx_s.at[j]], rows.at[b], gsems[b]).wait()

    def _starts(b, j):
        pltpu.async_copy(rows.at[b], acc.at[idx_d.at[j]], ssems[b], add=True)

    def _waits(b, j):
        pltpu.make_async_copy(rows.at[b], acc.at[idx_d.at[j]], ssems[b]).wait()

    @pl.loop(0, jnp.where(c == 0, NG_FAST, NG_SLOW))
    def _group(g):
        pltpu.sync_copy(src_hbm.at[wid, pl.ds(g * GC, GC)], idx_s)
        pltpu.sync_copy(dst_hbm.at[wid, pl.ds(g * GC, GC)], idx_d)
        for b in range(NBUF):
            _startg(b, b)

        @pl.loop(0, GC - NBUF, step=NBUF)
        def _chunk(j):
            for b in range(NBUF):
                _waitg(b, j + b)
                _starts(b, j + b)
            for b in range(NBUF):
                _waits(b, j + b)
                _startg(b, j + b + NBUF)

        for b in range(NBUF):
            _waitg(b, GC - NBUF + b)
            _starts(b, GC - NBUF + b)
        for b in range(NBUF):
            _waits(b, GC - NBUF + b)

    plsc.subcore_barrier()

    pltpu.sync_copy(acc.at[pl.ds(s * RPS, RPS)],
                    out_hbm.at[c, pl.ds(s * RPS, RPS)])


_segsum = pl.kernel(
    _segsum_body,
    out_type=jax.ShapeDtypeStruct((NC, N_PAD, D), jnp.float32),
    mesh=plsc.VectorSubcoreMesh(core_axis_name="c", subcore_axis_name="s"),
    scratch_types=[
        pltpu.VMEM((GC, C), jnp.int32),
        pltpu.VMEM((GC, C), jnp.int32),
        pltpu.VMEM((NBUF, C, D), jnp.float32),
        pltpu.VMEM_SHARED((N_PAD, D), jnp.float32),
    ] + [pltpu.SemaphoreType.DMA] * 8,
)


def _bn(z, g, b):
    m = jnp.mean(z, axis=0, keepdims=True)
    zc = z - m
    v = jnp.mean(zc * zc, axis=0, keepdims=True)
    return g * zc * jax.lax.rsqrt(v + 1e-5) + b


def _mlp(h_ref, p_ref, W1_ref, b1_ref, g1_ref, be1_ref, W2_ref, b2_ref,
         gbn_ref, bbn_ref, final_relu):
    z = h_ref[...] + p_ref[0, :N, :] + p_ref[1, :N, :]
    z = jnp.dot(z, W1_ref[...], preferred_element_type=jnp.float32) + b1_ref[...]
    z = _bn(z, g1_ref[...], be1_ref[...])
    z = jnp.maximum(z, 0.0)
    z = jnp.dot(z, W2_ref[...], preferred_element_type=jnp.float32) + b2_ref[...]
    z = _bn(z, gbn_ref[...], bbn_ref[...])
    if final_relu:
        z = jnp.maximum(z, 0.0)
    return z


def _dense0_body(h_ref, p_ref, W1_ref, b1_ref, g1_ref, be1_ref,
                 W2_ref, b2_ref, gbn_ref, bbn_ref, out_ref):
    out_ref[...] = _mlp(h_ref, p_ref, W1_ref, b1_ref, g1_ref, be1_ref,
                        W2_ref, b2_ref, gbn_ref, bbn_ref, final_relu=True)


def _dense1_body(idx_ref, h_ref, p_ref, W1_ref, b1_ref, g1_ref, be1_ref,
                 W2_ref, b2_ref, gbn_ref, bbn_ref, out_ref, sel_ref):
    out_ref[...] = _mlp(h_ref, p_ref, W1_ref, b1_ref, g1_ref, be1_ref,
                        W2_ref, b2_ref, gbn_ref, bbn_ref, final_relu=False)

    def _sel(j, carry):
        r = idx_ref[j]
        sel_ref[pl.ds(j, 1), :] = out_ref[pl.ds(r, 1), :]
        return carry
    lax.fori_loop(0, NG, _sel, 0)


_VSPEC = pl.BlockSpec(memory_space=pltpu.MemorySpace.VMEM)

_dense0 = pl.pallas_call(
    _dense0_body,
    out_shape=jax.ShapeDtypeStruct((N, D), jnp.float32),
    in_specs=[_VSPEC] * 10,
    out_specs=_VSPEC,
)

_dense1 = pl.pallas_call(
    _dense1_body,
    out_shape=(jax.ShapeDtypeStruct((N, D), jnp.float32),
               jax.ShapeDtypeStruct((NG, D), jnp.float32)),
    in_specs=[pl.BlockSpec(memory_space=pltpu.MemorySpace.SMEM)] + [_VSPEC] * 10,
    out_specs=(_VSPEC, _VSPEC),
)


def kernel(x, edge_index, edge_attr, batch, index,
           W1_0, b1_0, g1_0, be1_0, W2_0, b2_0, gbn_0, bbn_0,
           W1_1, b1_1, g1_1, be1_1, W2_1, b2_1, gbn_1, bbn_1):
    src = edge_index[0]
    dst = edge_index[1]
    pad = E_PAD - E
    sink = N + (jnp.arange(pad, dtype=jnp.int32) % (N_PAD - N))
    src_f = jnp.concatenate([src, jnp.zeros((pad,), jnp.int32)])
    dst_f = jnp.concatenate([dst, sink])
    nfast = NS * CH_FAST * C

    def _split(e):
        heavy = e[:nfast].reshape(NS, CH_FAST, C)
        if CH_SLOW:
            light = e[nfast:].reshape(NS, CH_SLOW, C)
            light = jnp.pad(light, ((0, 0), (0, CH_FAST - CH_SLOW), (0, 0)))
        else:
            light = jnp.zeros_like(heavy)
        return jnp.concatenate([heavy, light], axis=0)

    src_p = _split(src_f)
    dst_p = _split(dst_f)

    r1 = lambda a: a.reshape(1, D)
    p0 = _segsum(x, src_p, dst_p)
    h1 = _dense0(x, p0, W1_0, r1(b1_0), r1(g1_0), r1(be1_0),
                 W2_0, r1(b2_0), r1(gbn_0), r1(bbn_0))
    p1 = _segsum(h1, src_p, dst_p)
    h2, sel = _dense1(index, h1, p1, W1_1, r1(b1_1), r1(g1_1), r1(be1_1),
                      W2_1, r1(b2_1), r1(gbn_1), r1(bbn_1))
    return (h2, sel)
